# Initial kernel scaffold; baseline (speedup 1.0000x reference)
import jax
import jax.numpy as jnp
from jax.experimental import pallas as pl


def kernel(phase_indices, mag_indices, phase_cos_table, mag_exp_sin_table):
    raise NotImplementedError("write your pallas kernel here")


# speedup vs baseline: 137.3105x; 137.3105x over previous
import jax
import jax.numpy as jnp
from jax import lax
from jax.experimental import pallas as pl
from jax.experimental.pallas import tpu as pltpu
from jax.experimental.pallas import tpu_sc as plsc

N = 64
M = 1024
NC = 2
NS = 16
LANES = 16
NW = NC * NS

TOTAL = 16384 * 200
PER_W = TOTAL // NW
NCHUNK = 10
C = PER_W // NCHUNK
VECS = C // LANES


def _sc_body(pidx, midx, pt, mt, out, pt_v, mt_v, pidx_v, midx_v, out_v):
    wid = lax.axis_index("s") * NC + lax.axis_index("c")
    base = wid * PER_W
    pltpu.sync_copy(pt, pt_v)
    pltpu.sync_copy(mt, mt_v)

    def chunk_body(c, carry):
        off = base + c * C
        pltpu.sync_copy(pidx.at[pl.ds(off, C)], pidx_v)
        pltpu.sync_copy(midx.at[pl.ds(off, C)], midx_v)

        def vbody(i, carry2):
            s = pl.ds(i * LANES, LANES)
            piv = jnp.minimum(jnp.maximum(pidx_v[s], 0), N - 1)
            miv = jnp.minimum(jnp.maximum(midx_v[s], 0), M - 1)
            cv = plsc.load_gather(pt_v, [piv])
            mv = plsc.load_gather(mt_v, [miv])
            out_v[s] = cv * mv
            return carry2

        lax.fori_loop(0, VECS, vbody, 0)
        pltpu.sync_copy(out_v, out.at[pl.ds(off, C)])
        return carry

    lax.fori_loop(0, NCHUNK, chunk_body, 0)


def kernel(phase_indices, mag_indices, phase_cos_table, mag_exp_sin_table):
    B, L = phase_indices.shape
    pidx = phase_indices.reshape(-1).astype(jnp.int32)
    midx = mag_indices.reshape(-1).astype(jnp.int32)
    pt = phase_cos_table.astype(jnp.float32)
    mt = mag_exp_sin_table.astype(jnp.float32)
    mesh = plsc.VectorSubcoreMesh(core_axis_name="c", subcore_axis_name="s")
    out = pl.kernel(
        _sc_body,
        mesh=mesh,
        compiler_params=pltpu.CompilerParams(needs_layout_passes=False),
        out_type=jax.ShapeDtypeStruct((TOTAL,), jnp.float32),
        scratch_types=[
            pltpu.VMEM((N,), jnp.float32),
            pltpu.VMEM((M,), jnp.float32),
            pltpu.VMEM((C,), jnp.int32),
            pltpu.VMEM((C,), jnp.int32),
            pltpu.VMEM((C,), jnp.float32),
        ],
    )(pidx, midx, pt, mt)
    return out.reshape(B, L)


# speedup vs baseline: 179.6681x; 1.3085x over previous
import jax
import jax.numpy as jnp
from jax import lax
from jax.experimental import pallas as pl
from jax.experimental.pallas import tpu as pltpu
from jax.experimental.pallas import tpu_sc as plsc

N = 64
M = 1024
NC = 2
NS = 16
LANES = 16
NW = NC * NS

TOTAL = 16384 * 200
PER_W = TOTAL // NW
NCHUNK = 10
C = PER_W // NCHUNK
VECS = C // LANES


def _sc_body(pidx, midx, pt, mt, out,
             pt_v, mt_v,
             pidx_v0, pidx_v1, midx_v0, midx_v1, out_v0, out_v1,
             insem0, insem1, outsem0, outsem1):
    wid = lax.axis_index("s") * NC + lax.axis_index("c")
    base = wid * PER_W

    pbufs = (pidx_v0, pidx_v1)
    mbufs = (midx_v0, midx_v1)
    obufs = (out_v0, out_v1)
    insems = (insem0, insem1)
    outsems = (outsem0, outsem1)

    def start_in(c, b):
        off = base + c * C
        dp = pltpu.async_copy(pidx.at[pl.ds(off, C)], pbufs[b], insems[b])
        dm = pltpu.async_copy(midx.at[pl.ds(off, C)], mbufs[b], insems[b])
        return dp, dm

    din = {0: start_in(0, 0)}
    pltpu.sync_copy(pt, pt_v)
    pltpu.sync_copy(mt, mt_v)

    dout = {}
    for c in range(NCHUNK):
        b = c % 2
        if c + 1 < NCHUNK:
            din[c + 1] = start_in(c + 1, 1 - b)
        dp, dm = din.pop(c)
        dp.wait()
        dm.wait()
        if c >= 2:
            dout[b].wait()

        pv, mv, ov = pbufs[b], mbufs[b], obufs[b]

        @plsc.parallel_loop(0, VECS, unroll=8)
        def _(i):
            s = pl.ds(i * LANES, LANES)
            piv = jnp.minimum(jnp.maximum(pv[s], 0), N - 1)
            miv = jnp.minimum(jnp.maximum(mv[s], 0), M - 1)
            ov[s] = plsc.load_gather(pt_v, [piv]) * plsc.load_gather(mt_v, [miv])

        off = base + c * C
        dout[b] = pltpu.async_copy(ov, out.at[pl.ds(off, C)], outsems[b])

    dout[0].wait()
    dout[1].wait()


def kernel(phase_indices, mag_indices, phase_cos_table, mag_exp_sin_table):
    B, L = phase_indices.shape
    pidx = phase_indices.reshape(-1).astype(jnp.int32)
    midx = mag_indices.reshape(-1).astype(jnp.int32)
    pt = phase_cos_table.astype(jnp.float32)
    mt = mag_exp_sin_table.astype(jnp.float32)
    mesh = plsc.VectorSubcoreMesh(core_axis_name="c", subcore_axis_name="s")
    out = pl.kernel(
        _sc_body,
        mesh=mesh,
        compiler_params=pltpu.CompilerParams(needs_layout_passes=False),
        out_type=jax.ShapeDtypeStruct((TOTAL,), jnp.float32),
        scratch_types=[
            pltpu.VMEM((N,), jnp.float32),
            pltpu.VMEM((M,), jnp.float32),
            pltpu.VMEM((C,), jnp.int32),
            pltpu.VMEM((C,), jnp.int32),
            pltpu.VMEM((C,), jnp.int32),
            pltpu.VMEM((C,), jnp.int32),
            pltpu.VMEM((C,), jnp.float32),
            pltpu.VMEM((C,), jnp.float32),
            pltpu.SemaphoreType.DMA,
            pltpu.SemaphoreType.DMA,
            pltpu.SemaphoreType.DMA,
            pltpu.SemaphoreType.DMA,
        ],
    )(pidx, midx, pt, mt)
    return out.reshape(B, L)


# speedup vs baseline: 301.5665x; 1.6785x over previous
import jax
import jax.numpy as jnp
from jax import lax
from jax.experimental import pallas as pl
from jax.experimental.pallas import tpu as pltpu
from jax.experimental.pallas import tpu_sc as plsc

N = 64
M = 1024
NC = 2
NS = 16
LANES = 16
NW = NC * NS

B_ROWS = 16384
ROW = 200
ROWS_PER_W = B_ROWS // NW
RB = 64
NCHUNK = ROWS_PER_W // RB
JS = tuple(range(0, ROW - LANES + 1, LANES)) + (ROW - LANES,)


def _sc_body(pidx, midx, pt, mt, out,
             pt_v, mt_v,
             pidx_v0, pidx_v1, midx_v0, midx_v1, out_v0, out_v1,
             insem0, insem1, outsem0, outsem1):
    wid = lax.axis_index("s") * NC + lax.axis_index("c")
    row0 = wid * ROWS_PER_W

    pbufs = (pidx_v0, pidx_v1)
    mbufs = (midx_v0, midx_v1)
    obufs = (out_v0, out_v1)
    insems = (insem0, insem1)
    outsems = (outsem0, outsem1)

    def start_in(c, b):
        r = row0 + c * RB
        dp = pltpu.async_copy(pidx.at[pl.ds(r, RB), :], pbufs[b], insems[b])
        dm = pltpu.async_copy(midx.at[pl.ds(r, RB), :], mbufs[b], insems[b])
        return dp, dm

    din = {0: start_in(0, 0)}
    pltpu.sync_copy(pt, pt_v)
    pltpu.sync_copy(mt, mt_v)

    dout = {}
    for c in range(NCHUNK):
        b = c % 2
        if c + 1 < NCHUNK:
            din[c + 1] = start_in(c + 1, 1 - b)
        dp, dm = din.pop(c)
        dp.wait()
        dm.wait()
        if c >= 2:
            dout[b].wait()

        pv, mv, ov = pbufs[b], mbufs[b], obufs[b]

        @plsc.parallel_loop(0, RB, unroll=2)
        def _(r):
            for js in JS:
                s = pl.ds(js, LANES)
                piv = jnp.minimum(jnp.maximum(pv[r, s], 0), N - 1)
                miv = jnp.minimum(jnp.maximum(mv[r, s], 0), M - 1)
                ov[r, s] = plsc.load_gather(pt_v, [piv]) * plsc.load_gather(mt_v, [miv])

        r = row0 + c * RB
        dout[b] = pltpu.async_copy(ov, out.at[pl.ds(r, RB), :], outsems[b])

    dout[0].wait()
    dout[1].wait()


def kernel(phase_indices, mag_indices, phase_cos_table, mag_exp_sin_table):
    pidx = phase_indices.astype(jnp.int32)
    midx = mag_indices.astype(jnp.int32)
    pt = phase_cos_table.astype(jnp.float32)
    mt = mag_exp_sin_table.astype(jnp.float32)
    mesh = plsc.VectorSubcoreMesh(core_axis_name="c", subcore_axis_name="s")
    out = pl.kernel(
        _sc_body,
        mesh=mesh,
        compiler_params=pltpu.CompilerParams(needs_layout_passes=False),
        out_type=jax.ShapeDtypeStruct((B_ROWS, ROW), jnp.float32),
        scratch_types=[
            pltpu.VMEM((N,), jnp.float32),
            pltpu.VMEM((M,), jnp.float32),
            pltpu.VMEM((RB, ROW), jnp.int32),
            pltpu.VMEM((RB, ROW), jnp.int32),
            pltpu.VMEM((RB, ROW), jnp.int32),
            pltpu.VMEM((RB, ROW), jnp.int32),
            pltpu.VMEM((RB, ROW), jnp.float32),
            pltpu.VMEM((RB, ROW), jnp.float32),
            pltpu.SemaphoreType.DMA,
            pltpu.SemaphoreType.DMA,
            pltpu.SemaphoreType.DMA,
            pltpu.SemaphoreType.DMA,
        ],
    )(pidx, midx, pt, mt)
    return out


# speedup vs baseline: 485.5098x; 1.6100x over previous
import jax
import jax.numpy as jnp
from jax import lax
from jax.experimental import pallas as pl
from jax.experimental.pallas import tpu as pltpu
from jax.experimental.pallas import tpu_sc as plsc

N = 64
M = 1024
NC = 2
NS = 16
LANES = 16
NW = NC * NS

B_ROWS = 16384
ROW = 200
COLS_PER_W = B_ROWS // NW
RB = 40
NCHUNK = ROW // RB
CVECS = COLS_PER_W // LANES


def _sc_body(pidx, midx, pt, mt, out,
             pt_v, mt_v,
             pidx_v0, pidx_v1, midx_v0, midx_v1, out_v0, out_v1,
             insem0, insem1, outsem0, outsem1):
    wid = lax.axis_index("s") * NC + lax.axis_index("c")
    col0 = wid * COLS_PER_W

    pbufs = (pidx_v0, pidx_v1)
    mbufs = (midx_v0, midx_v1)
    obufs = (out_v0, out_v1)
    insems = (insem0, insem1)
    outsems = (outsem0, outsem1)

    def start_in(c, b):
        r = c * RB
        src_p = pidx.at[pl.ds(r, RB), pl.ds(col0, COLS_PER_W)]
        src_m = midx.at[pl.ds(r, RB), pl.ds(col0, COLS_PER_W)]
        dp = pltpu.async_copy(src_p, pbufs[b], insems[b])
        dm = pltpu.async_copy(src_m, mbufs[b], insems[b])
        return dp, dm

    din = {0: start_in(0, 0)}
    pltpu.sync_copy(pt, pt_v)
    pltpu.sync_copy(mt, mt_v)

    dout = {}
    for c in range(NCHUNK):
        b = c % 2
        if c + 1 < NCHUNK:
            din[c + 1] = start_in(c + 1, 1 - b)
        dp, dm = din.pop(c)
        dp.wait()
        dm.wait()
        if c >= 2:
            dout[b].wait()

        pv, mv, ov = pbufs[b], mbufs[b], obufs[b]

        @plsc.parallel_loop(0, RB, unroll=2)
        def _(r):
            for j in range(CVECS):
                s = pl.ds(j * LANES, LANES)
                piv = jnp.minimum(jnp.maximum(pv[r, s], 0), N - 1)
                miv = jnp.minimum(jnp.maximum(mv[r, s], 0), M - 1)
                ov[r, s] = plsc.load_gather(pt_v, [piv]) * plsc.load_gather(mt_v, [miv])

        r = c * RB
        dout[b] = pltpu.async_copy(
            ov, out.at[pl.ds(r, RB), pl.ds(col0, COLS_PER_W)], outsems[b])

    dout[0].wait()
    dout[1].wait()


def kernel(phase_indices, mag_indices, phase_cos_table, mag_exp_sin_table):
    pidx = phase_indices.astype(jnp.int32).T
    midx = mag_indices.astype(jnp.int32).T
    pt = phase_cos_table.astype(jnp.float32)
    mt = mag_exp_sin_table.astype(jnp.float32)
    mesh = plsc.VectorSubcoreMesh(core_axis_name="c", subcore_axis_name="s")
    out = pl.kernel(
        _sc_body,
        mesh=mesh,
        compiler_params=pltpu.CompilerParams(needs_layout_passes=False),
        out_type=jax.ShapeDtypeStruct((ROW, B_ROWS), jnp.float32),
        scratch_types=[
            pltpu.VMEM((N,), jnp.float32),
            pltpu.VMEM((M,), jnp.float32),
            pltpu.VMEM((RB, COLS_PER_W), jnp.int32),
            pltpu.VMEM((RB, COLS_PER_W), jnp.int32),
            pltpu.VMEM((RB, COLS_PER_W), jnp.int32),
            pltpu.VMEM((RB, COLS_PER_W), jnp.int32),
            pltpu.VMEM((RB, COLS_PER_W), jnp.float32),
            pltpu.VMEM((RB, COLS_PER_W), jnp.float32),
            pltpu.SemaphoreType.DMA,
            pltpu.SemaphoreType.DMA,
            pltpu.SemaphoreType.DMA,
            pltpu.SemaphoreType.DMA,
        ],
    )(pidx, midx, pt, mt)
    return out.T


# speedup vs baseline: 519.8787x; 1.0708x over previous
import jax
import jax.numpy as jnp
from jax import lax
from jax.experimental import pallas as pl
from jax.experimental.pallas import tpu as pltpu
from jax.experimental.pallas import tpu_sc as plsc

N = 64
M = 1024
NC = 2
NS = 16
LANES = 16
NW = NC * NS

B_ROWS = 16384
ROW = 200
COLS_PER_W = B_ROWS // NW
RB = 40
NCHUNK = ROW // RB
CVECS = COLS_PER_W // LANES


def _sc_body(pidx, midx, pt, mt, out,
             pt_v, mt_v,
             pidx_v0, pidx_v1, midx_v0, midx_v1, out_v0, out_v1,
             insem0, insem1, outsem0, outsem1, tsem):
    wid = lax.axis_index("s") * NC + lax.axis_index("c")
    col0 = wid * COLS_PER_W

    pbufs = (pidx_v0, pidx_v1)
    mbufs = (midx_v0, midx_v1)
    obufs = (out_v0, out_v1)
    insems = (insem0, insem1)
    outsems = (outsem0, outsem1)

    def start_in(c, b):
        r = c * RB
        src_p = pidx.at[pl.ds(r, RB), pl.ds(col0, COLS_PER_W)]
        src_m = midx.at[pl.ds(r, RB), pl.ds(col0, COLS_PER_W)]
        dp = pltpu.async_copy(src_p, pbufs[b], insems[b])
        dm = pltpu.async_copy(src_m, mbufs[b], insems[b])
        return dp, dm

    din = {0: start_in(0, 0)}
    dt0 = pltpu.async_copy(pt, pt_v, tsem)
    dt1 = pltpu.async_copy(mt, mt_v, tsem)
    din[1] = start_in(1, 1)
    dt0.wait()
    dt1.wait()

    dout = {}
    for c in range(NCHUNK):
        b = c % 2
        dp, dm = din.pop(c)
        dp.wait()
        dm.wait()
        if c >= 2:
            dout[b].wait()

        pv, mv, ov = pbufs[b], mbufs[b], obufs[b]

        @plsc.parallel_loop(0, RB, unroll=1)
        def _(r):
            for j in range(CVECS):
                s = pl.ds(j * LANES, LANES)
                piv = jnp.minimum(jnp.maximum(pv[r, s], 0), N - 1)
                miv = jnp.minimum(jnp.maximum(mv[r, s], 0), M - 1)
                ov[r, s] = plsc.load_gather(pt_v, [piv]) * plsc.load_gather(mt_v, [miv])

        r = c * RB
        dout[b] = pltpu.async_copy(
            ov, out.at[pl.ds(r, RB), pl.ds(col0, COLS_PER_W)], outsems[b])
        if c + 2 < NCHUNK:
            din[c + 2] = start_in(c + 2, b)

    dout[0].wait()
    dout[1].wait()


def kernel(phase_indices, mag_indices, phase_cos_table, mag_exp_sin_table):
    pidx = phase_indices.astype(jnp.int32).T
    midx = mag_indices.astype(jnp.int32).T
    pt = phase_cos_table.astype(jnp.float32)
    mt = mag_exp_sin_table.astype(jnp.float32)
    mesh = plsc.VectorSubcoreMesh(core_axis_name="c", subcore_axis_name="s")
    out = pl.kernel(
        _sc_body,
        mesh=mesh,
        compiler_params=pltpu.CompilerParams(needs_layout_passes=False),
        out_type=jax.ShapeDtypeStruct((ROW, B_ROWS), jnp.float32),
        scratch_types=[
            pltpu.VMEM((N,), jnp.float32),
            pltpu.VMEM((M,), jnp.float32),
            pltpu.VMEM((RB, COLS_PER_W), jnp.int32),
            pltpu.VMEM((RB, COLS_PER_W), jnp.int32),
            pltpu.VMEM((RB, COLS_PER_W), jnp.int32),
            pltpu.VMEM((RB, COLS_PER_W), jnp.int32),
            pltpu.VMEM((RB, COLS_PER_W), jnp.float32),
            pltpu.VMEM((RB, COLS_PER_W), jnp.float32),
            pltpu.SemaphoreType.DMA,
            pltpu.SemaphoreType.DMA,
            pltpu.SemaphoreType.DMA,
            pltpu.SemaphoreType.DMA,
            pltpu.SemaphoreType.DMA,
        ],
    )(pidx, midx, pt, mt)
    return out.T
